# baseline (speedup 1.0000x reference)
import functools

import jax
import jax.numpy as jnp
from jax import lax
from jax.experimental import pallas as pl
from jax.experimental.pallas import tpu as pltpu
from jax.experimental.pallas import tpu_sc as plsc

_SMOOTHING = 0.1
_IGNORE_INDEX = 0

_N_SC = 768
_SC_COLS = 96000
_TRPW = 3
_CHUNK = 6400
_TC_ROWS_PER_BLOCK = 32


def _loss_rows_kernel(pred_ref, tgt_ref, out_ref, *, vocab):
    x = pred_ref[...]
    t = tgt_ref[...]
    m = jnp.max(x, axis=-1, keepdims=True)
    s = jnp.sum(jnp.exp(x - m), axis=-1, keepdims=True)
    lse = m + jnp.log(s)
    sum_x = jnp.sum(x, axis=-1, keepdims=True)
    lane = jax.lax.broadcasted_iota(jnp.int32, x.shape, 1)
    pred_t = jnp.sum(jnp.where(lane == t, x, 0.0), axis=-1, keepdims=True)
    eps = _SMOOTHING / (vocab - 2)
    conf = 1.0 - _SMOOTHING
    sum_logp = sum_x - vocab * lse
    logp_t = pred_t - lse
    loss = -(eps * sum_logp + (conf - eps) * logp_t)
    out_ref[...] = jnp.where(t == _IGNORE_INDEX, 0.0, loss)


def _tc_main(pred, tgt2d, vocab):
    n = pred.shape[0]
    r = _TC_ROWS_PER_BLOCK
    steps = (n - _N_SC) // r
    off = _N_SC // r
    return pl.pallas_call(
        functools.partial(_loss_rows_kernel, vocab=vocab),
        grid=(steps,),
        in_specs=[
            pl.BlockSpec((r, vocab), lambda i: (i + off, 0)),
            pl.BlockSpec((r, 1), lambda i: (i + off, 0)),
        ],
        out_specs=pl.BlockSpec((r, 1), lambda i: (i, 0)),
        out_shape=jax.ShapeDtypeStruct((n - _N_SC, 1), jnp.float32),
    )(pred, tgt2d)


def _tail_kernel(pred_ref, tgt_ref, m_ref, s_ref, sx_ref, pt_ref, *, col0):
    x = pred_ref[...]
    t = tgt_ref[...]
    m = jnp.max(x, axis=-1, keepdims=True)
    s = jnp.sum(jnp.exp(x - m), axis=-1, keepdims=True)
    sx = jnp.sum(x, axis=-1, keepdims=True)
    col = col0 + jax.lax.broadcasted_iota(jnp.int32, x.shape, 1)
    pt = jnp.sum(jnp.where(col == t, x, 0.0), axis=-1, keepdims=True)
    m_ref[...] = m
    s_ref[...] = s
    sx_ref[...] = sx
    pt_ref[...] = pt


def _tc_tail(pred_tail, tgt2d, vocab):
    r = 64
    tail = vocab - _SC_COLS
    steps = _N_SC // r
    out = jax.ShapeDtypeStruct((_N_SC, 1), jnp.float32)
    return pl.pallas_call(
        functools.partial(_tail_kernel, col0=_SC_COLS),
        grid=(steps,),
        in_specs=[
            pl.BlockSpec((r, tail), lambda i: (i, 0)),
            pl.BlockSpec((r, 1), lambda i: (i, 0)),
        ],
        out_specs=[pl.BlockSpec((r, 1), lambda i: (i, 0))] * 4,
        out_shape=[out] * 4,
    )(pred_tail, tgt2d)


def _make_sc_part(vocab):
    ntiles = _CHUNK // 128
    nchunks = _SC_COLS // _CHUNK
    rpw = _TRPW * 8
    mesh = plsc.VectorSubcoreMesh(core_axis_name="c", subcore_axis_name="s")
    lanes_out = jax.ShapeDtypeStruct((_N_SC * 16,), jnp.float32)

    @functools.partial(
        pl.kernel,
        mesh=mesh,
        out_type=(lanes_out, lanes_out, lanes_out, lanes_out),
        scratch_types=[
            pltpu.VMEM((8, _CHUNK), jnp.float32),
            pltpu.VMEM((8, 128), jnp.float32),
            pltpu.VMEM((rpw,), jnp.int32),
            pltpu.VMEM((rpw * 16,), jnp.float32),
            pltpu.VMEM((rpw * 16,), jnp.float32),
            pltpu.VMEM((rpw * 16,), jnp.float32),
            pltpu.VMEM((rpw * 16,), jnp.float32),
        ],
        compiler_params=pltpu.CompilerParams(use_tc_tiling_on_sc=True),
    )
    def sc_stats(pred_hbm, tgt_hbm, m_hbm, s_hbm, sx_hbm, pt_hbm,
                 chunk_v, ttile_v, tgt_v, m_st, s_st, sx_st, pt_st):
        nc = 2
        wid = lax.axis_index("s") * nc + lax.axis_index("c")
        base = wid * rpw
        pltpu.sync_copy(tgt_hbm.at[pl.ds(base, rpw)], tgt_v)
        lane = lax.broadcasted_iota(jnp.int32, (16,), 0)
        zeros = jnp.zeros((16,), jnp.float32)
        ninf = jnp.full((16,), -jnp.inf, jnp.float32)

        for tr in range(_TRPW):
            row0 = base + tr * 8

            def chunk_body(c, carry):
                ms, ss, sxs = carry
                pltpu.sync_copy(
                    pred_hbm.at[pl.ds(row0, 8), pl.ds(c * _CHUNK, _CHUNK)],
                    chunk_v)
                ms_n, ss_n, sxs_n = [], [], []
                for r8 in range(8):
                    def b1(i, mv, r8=r8):
                        for g in range(8):
                            mv = jnp.maximum(
                                mv, chunk_v[r8, pl.ds(i * 128 + g * 16, 16)])
                        return mv
                    cm = lax.fori_loop(0, ntiles, b1, ninf)
                    m_new = jnp.maximum(ms[r8], cm)

                    def b2(i, car, r8=r8, m_new=m_new):
                        sv, sxv = car
                        for g in range(8):
                            v = chunk_v[r8, pl.ds(i * 128 + g * 16, 16)]
                            sv = sv + jnp.exp(v - m_new)
                            sxv = sxv + v
                        return sv, sxv
                    sv, sxv = lax.fori_loop(0, ntiles, b2, (zeros, zeros))
                    ss_n.append(ss[r8] * jnp.exp(ms[r8] - m_new) + sv)
                    ms_n.append(m_new)
                    sxs_n.append(sxs[r8] + sxv)
                return tuple(ms_n), tuple(ss_n), tuple(sxs_n)

            ms, ss, sxs = lax.fori_loop(
                0, nchunks, chunk_body,
                ((ninf,) * 8, (zeros,) * 8, (zeros,) * 8))
            for r8 in range(8):
                ridx = tr * 8 + r8
                m_st[pl.ds(ridx * 16, 16)] = ms[r8]
                s_st[pl.ds(ridx * 16, 16)] = ss[r8]
                sx_st[pl.ds(ridx * 16, 16)] = sxs[r8]

        tlo = tgt_v[pl.ds(0, 16)]
        thi = tgt_v[pl.ds(8, 16)]
        for r in range(rpw):
            t = tlo[r] if r < 16 else thi[r - 8]
            row0 = base + (r // 8) * 8
            pt_st[pl.ds(r * 16, 16)] = zeros

            @pl.when(t < _SC_COLS)
            def _(t=t, r=r, row0=row0):
                pltpu.sync_copy(
                    pred_hbm.at[pl.ds(row0, 8),
                                pl.ds((t // 128) * 128, 128)], ttile_v)
                w = ttile_v[r % 8, pl.ds(((t % 128) // 16) * 16, 16)]
                pt_st[pl.ds(r * 16, 16)] = jnp.where(
                    lane == t % 16, w, 0.0)

        pltpu.sync_copy(m_st, m_hbm.at[pl.ds(base * 16, rpw * 16)])
        pltpu.sync_copy(s_st, s_hbm.at[pl.ds(base * 16, rpw * 16)])
        pltpu.sync_copy(sx_st, sx_hbm.at[pl.ds(base * 16, rpw * 16)])
        pltpu.sync_copy(pt_st, pt_hbm.at[pl.ds(base * 16, rpw * 16)])

    return sc_stats


def kernel(pred, target):
    n, vocab = pred.shape
    tgt = target.astype(jnp.int32)
    tgt2d = tgt.reshape(n, 1)
    eps = _SMOOTHING / (vocab - 2)
    conf = 1.0 - _SMOOTHING

    m, s, sx, pt = _make_sc_part(vocab)(pred, tgt)
    mt, st, sxt, ptt = _tc_tail(pred[:_N_SC, _SC_COLS:], tgt2d, vocab)
    tc_losses = _tc_main(pred, tgt2d, vocab)

    m = m.reshape(_N_SC, 16)
    s = s.reshape(_N_SC, 16)
    sx = sx.reshape(_N_SC, 16)
    mt = mt.reshape(_N_SC)
    st = st.reshape(_N_SC)
    sxt = sxt.reshape(_N_SC)
    ptt = ptt.reshape(_N_SC)
    row_max = jnp.maximum(jnp.max(m, axis=1), mt)
    row_s = (jnp.sum(s * jnp.exp(m - row_max[:, None]), axis=1)
             + st * jnp.exp(mt - row_max))
    lse = row_max + jnp.log(row_s)
    sum_x = jnp.sum(sx, axis=1) + sxt
    pt_row = jnp.sum(pt.reshape(_N_SC, 16), axis=1) + ptt
    sc_loss = -(eps * (sum_x - vocab * lse) + (conf - eps) * (pt_row - lse))
    sc_loss = jnp.where(tgt[:_N_SC] == _IGNORE_INDEX, 0.0, sc_loss)
    return (jnp.sum(sc_loss) + jnp.sum(tc_losses)) / n


# speedup vs baseline: 1.4277x; 1.4277x over previous
import functools

import jax
import jax.numpy as jnp
from jax import lax
from jax.experimental import pallas as pl
from jax.experimental.pallas import tpu as pltpu
from jax.experimental.pallas import tpu_sc as plsc

_SMOOTHING = 0.1
_IGNORE_INDEX = 0

_N_SC = 256
_SC_COLS = 96000
_TRPW = 1
_CHUNK = 6400
_TC_ROWS_PER_BLOCK = 32


def _loss_rows_kernel(pred_ref, tgt_ref, out_ref, *, vocab):
    x = pred_ref[...]
    t = tgt_ref[...]
    m = jnp.max(x, axis=-1, keepdims=True)
    s = jnp.sum(jnp.exp(x - m), axis=-1, keepdims=True)
    lse = m + jnp.log(s)
    sum_x = jnp.sum(x, axis=-1, keepdims=True)
    lane = jax.lax.broadcasted_iota(jnp.int32, x.shape, 1)
    pred_t = jnp.sum(jnp.where(lane == t, x, 0.0), axis=-1, keepdims=True)
    eps = _SMOOTHING / (vocab - 2)
    conf = 1.0 - _SMOOTHING
    sum_logp = sum_x - vocab * lse
    logp_t = pred_t - lse
    loss = -(eps * sum_logp + (conf - eps) * logp_t)
    out_ref[...] = jnp.where(t == _IGNORE_INDEX, 0.0, loss)


def _tc_main(pred, tgt2d, vocab):
    n = pred.shape[0]
    r = _TC_ROWS_PER_BLOCK
    steps = (n - _N_SC) // r
    off = _N_SC // r
    return pl.pallas_call(
        functools.partial(_loss_rows_kernel, vocab=vocab),
        grid=(steps,),
        in_specs=[
            pl.BlockSpec((r, vocab), lambda i: (i + off, 0)),
            pl.BlockSpec((r, 1), lambda i: (i + off, 0)),
        ],
        out_specs=pl.BlockSpec((r, 1), lambda i: (i, 0)),
        out_shape=jax.ShapeDtypeStruct((n - _N_SC, 1), jnp.float32),
    )(pred, tgt2d)


def _tail_kernel(pred_ref, tgt_ref, m_ref, s_ref, sx_ref, pt_ref, *, col0):
    x = pred_ref[...]
    t = tgt_ref[...]
    m = jnp.max(x, axis=-1, keepdims=True)
    s = jnp.sum(jnp.exp(x - m), axis=-1, keepdims=True)
    sx = jnp.sum(x, axis=-1, keepdims=True)
    col = col0 + jax.lax.broadcasted_iota(jnp.int32, x.shape, 1)
    pt = jnp.sum(jnp.where(col == t, x, 0.0), axis=-1, keepdims=True)
    m_ref[...] = m
    s_ref[...] = s
    sx_ref[...] = sx
    pt_ref[...] = pt


def _tc_tail(pred_tail, tgt2d, vocab):
    r = 64
    tail = vocab - _SC_COLS
    steps = _N_SC // r
    out = jax.ShapeDtypeStruct((_N_SC, 1), jnp.float32)
    return pl.pallas_call(
        functools.partial(_tail_kernel, col0=_SC_COLS),
        grid=(steps,),
        in_specs=[
            pl.BlockSpec((r, tail), lambda i: (i, 0)),
            pl.BlockSpec((r, 1), lambda i: (i, 0)),
        ],
        out_specs=[pl.BlockSpec((r, 1), lambda i: (i, 0))] * 4,
        out_shape=[out] * 4,
    )(pred_tail, tgt2d)


def _make_sc_part(vocab):
    ntiles = _CHUNK // 128
    nchunks = _SC_COLS // _CHUNK
    rpw = _TRPW * 8
    mesh = plsc.VectorSubcoreMesh(core_axis_name="c", subcore_axis_name="s")
    lanes_out = jax.ShapeDtypeStruct((_N_SC * 16,), jnp.float32)

    @functools.partial(
        pl.kernel,
        mesh=mesh,
        out_type=(lanes_out, lanes_out, lanes_out, lanes_out),
        scratch_types=[
            pltpu.VMEM((8, _CHUNK), jnp.float32),
            pltpu.VMEM((8, 128), jnp.float32),
            pltpu.VMEM((max(rpw, 16),), jnp.int32),
            pltpu.VMEM((rpw * 16,), jnp.float32),
            pltpu.VMEM((rpw * 16,), jnp.float32),
            pltpu.VMEM((rpw * 16,), jnp.float32),
            pltpu.VMEM((rpw * 16,), jnp.float32),
        ],
        compiler_params=pltpu.CompilerParams(use_tc_tiling_on_sc=True),
    )
    def sc_stats(pred_hbm, tgt_hbm, m_hbm, s_hbm, sx_hbm, pt_hbm,
                 chunk_v, ttile_v, tgt_v, m_st, s_st, sx_st, pt_st):
        nc = 2
        wid = lax.axis_index("s") * nc + lax.axis_index("c")
        base = wid * rpw
        pltpu.sync_copy(tgt_hbm.at[pl.ds(base, max(rpw, 16))], tgt_v)
        lane = lax.broadcasted_iota(jnp.int32, (16,), 0)
        zeros = jnp.zeros((16,), jnp.float32)
        ninf = jnp.full((16,), -jnp.inf, jnp.float32)

        for tr in range(_TRPW):
            row0 = base + tr * 8

            def chunk_body(c, carry):
                ms, ss, sxs = carry
                pltpu.sync_copy(
                    pred_hbm.at[pl.ds(row0, 8), pl.ds(c * _CHUNK, _CHUNK)],
                    chunk_v)
                ms_n, ss_n, sxs_n = [], [], []
                for r8 in range(8):
                    def b1(i, mv, r8=r8):
                        for g in range(8):
                            mv = jnp.maximum(
                                mv, chunk_v[r8, pl.ds(i * 128 + g * 16, 16)])
                        return mv
                    cm = lax.fori_loop(0, ntiles, b1, ninf)
                    m_new = jnp.maximum(ms[r8], cm)

                    def b2(i, car, r8=r8, m_new=m_new):
                        sv, sxv = car
                        for g in range(8):
                            v = chunk_v[r8, pl.ds(i * 128 + g * 16, 16)]
                            sv = sv + jnp.exp(v - m_new)
                            sxv = sxv + v
                        return sv, sxv
                    sv, sxv = lax.fori_loop(0, ntiles, b2, (zeros, zeros))
                    ss_n.append(ss[r8] * jnp.exp(ms[r8] - m_new) + sv)
                    ms_n.append(m_new)
                    sxs_n.append(sxs[r8] + sxv)
                return tuple(ms_n), tuple(ss_n), tuple(sxs_n)

            ms, ss, sxs = lax.fori_loop(
                0, nchunks, chunk_body,
                ((ninf,) * 8, (zeros,) * 8, (zeros,) * 8))
            for r8 in range(8):
                ridx = tr * 8 + r8
                m_st[pl.ds(ridx * 16, 16)] = ms[r8]
                s_st[pl.ds(ridx * 16, 16)] = ss[r8]
                sx_st[pl.ds(ridx * 16, 16)] = sxs[r8]

        tlo = tgt_v[pl.ds(0, 16)]
        thi = tgt_v[pl.ds(rpw - 16, 16)] if rpw > 16 else tlo
        for r in range(rpw):
            t = tlo[r] if r < 16 else thi[r - (rpw - 16)]
            row0 = base + (r // 8) * 8
            pt_st[pl.ds(r * 16, 16)] = zeros

            @pl.when(t < _SC_COLS)
            def _(t=t, r=r, row0=row0):
                pltpu.sync_copy(
                    pred_hbm.at[pl.ds(row0, 8),
                                pl.ds((t // 128) * 128, 128)], ttile_v)
                w = ttile_v[r % 8, pl.ds(((t % 128) // 16) * 16, 16)]
                pt_st[pl.ds(r * 16, 16)] = jnp.where(
                    lane == t % 16, w, 0.0)

        pltpu.sync_copy(m_st, m_hbm.at[pl.ds(base * 16, rpw * 16)])
        pltpu.sync_copy(s_st, s_hbm.at[pl.ds(base * 16, rpw * 16)])
        pltpu.sync_copy(sx_st, sx_hbm.at[pl.ds(base * 16, rpw * 16)])
        pltpu.sync_copy(pt_st, pt_hbm.at[pl.ds(base * 16, rpw * 16)])

    return sc_stats


def kernel(pred, target):
    n, vocab = pred.shape
    tgt = target.astype(jnp.int32)
    tgt2d = tgt.reshape(n, 1)
    eps = _SMOOTHING / (vocab - 2)
    conf = 1.0 - _SMOOTHING

    m, s, sx, pt = _make_sc_part(vocab)(pred, tgt)
    mt, st, sxt, ptt = _tc_tail(pred[:_N_SC, _SC_COLS:], tgt2d, vocab)
    tc_losses = _tc_main(pred, tgt2d, vocab)

    m = m.reshape(_N_SC, 16)
    s = s.reshape(_N_SC, 16)
    sx = sx.reshape(_N_SC, 16)
    mt = mt.reshape(_N_SC)
    st = st.reshape(_N_SC)
    sxt = sxt.reshape(_N_SC)
    ptt = ptt.reshape(_N_SC)
    row_max = jnp.maximum(jnp.max(m, axis=1), mt)
    row_s = (jnp.sum(s * jnp.exp(m - row_max[:, None]), axis=1)
             + st * jnp.exp(mt - row_max))
    lse = row_max + jnp.log(row_s)
    sum_x = jnp.sum(sx, axis=1) + sxt
    pt_row = jnp.sum(pt.reshape(_N_SC, 16), axis=1) + ptt
    sc_loss = -(eps * (sum_x - vocab * lse) + (conf - eps) * (pt_row - lse))
    sc_loss = jnp.where(tgt[:_N_SC] == _IGNORE_INDEX, 0.0, sc_loss)
    return (jnp.sum(sc_loss) + jnp.sum(tc_losses)) / n


# speedup vs baseline: 1.4573x; 1.0207x over previous
import functools

import jax
import jax.numpy as jnp
from jax import lax
from jax.experimental import pallas as pl
from jax.experimental.pallas import tpu as pltpu
from jax.experimental.pallas import tpu_sc as plsc

_SMOOTHING = 0.1
_IGNORE_INDEX = 0

_N_SC = 256
_SC_COLS = 96000
_TRPW = 1
_CHUNK = 6400
_TC_ROWS_PER_BLOCK = 64


def _loss_rows_kernel(pred_ref, tgt_ref, out_ref, *, vocab):
    x = pred_ref[...]
    t = tgt_ref[...]
    m = jnp.max(x, axis=-1, keepdims=True)
    s = jnp.sum(jnp.exp(x - m), axis=-1, keepdims=True)
    lse = m + jnp.log(s)
    sum_x = jnp.sum(x, axis=-1, keepdims=True)
    lane = jax.lax.broadcasted_iota(jnp.int32, x.shape, 1)
    pred_t = jnp.sum(jnp.where(lane == t, x, 0.0), axis=-1, keepdims=True)
    eps = _SMOOTHING / (vocab - 2)
    conf = 1.0 - _SMOOTHING
    sum_logp = sum_x - vocab * lse
    logp_t = pred_t - lse
    loss = -(eps * sum_logp + (conf - eps) * logp_t)
    out_ref[...] = jnp.where(t == _IGNORE_INDEX, 0.0, loss)


def _tc_main(pred, tgt2d, vocab):
    n = pred.shape[0]
    r = _TC_ROWS_PER_BLOCK
    steps = (n - _N_SC) // r
    off = _N_SC // r
    return pl.pallas_call(
        functools.partial(_loss_rows_kernel, vocab=vocab),
        grid=(steps,),
        in_specs=[
            pl.BlockSpec((r, vocab), lambda i: (i + off, 0)),
            pl.BlockSpec((r, 1), lambda i: (i + off, 0)),
        ],
        out_specs=pl.BlockSpec((r, 1), lambda i: (i, 0)),
        out_shape=jax.ShapeDtypeStruct((n - _N_SC, 1), jnp.float32),
    )(pred, tgt2d)


def _tail_kernel(pred_ref, tgt_ref, m_ref, s_ref, sx_ref, pt_ref, *, col0):
    x = pred_ref[...]
    t = tgt_ref[...]
    m = jnp.max(x, axis=-1, keepdims=True)
    s = jnp.sum(jnp.exp(x - m), axis=-1, keepdims=True)
    sx = jnp.sum(x, axis=-1, keepdims=True)
    col = col0 + jax.lax.broadcasted_iota(jnp.int32, x.shape, 1)
    pt = jnp.sum(jnp.where(col == t, x, 0.0), axis=-1, keepdims=True)
    m_ref[...] = m
    s_ref[...] = s
    sx_ref[...] = sx
    pt_ref[...] = pt


def _tc_tail(pred_tail, tgt2d, vocab):
    r = 64
    tail = vocab - _SC_COLS
    steps = _N_SC // r
    out = jax.ShapeDtypeStruct((_N_SC, 1), jnp.float32)
    return pl.pallas_call(
        functools.partial(_tail_kernel, col0=_SC_COLS),
        grid=(steps,),
        in_specs=[
            pl.BlockSpec((r, tail), lambda i: (i, 0)),
            pl.BlockSpec((r, 1), lambda i: (i, 0)),
        ],
        out_specs=[pl.BlockSpec((r, 1), lambda i: (i, 0))] * 4,
        out_shape=[out] * 4,
    )(pred_tail, tgt2d)


def _make_sc_part(vocab):
    ntiles = _CHUNK // 128
    nchunks = _SC_COLS // _CHUNK
    rpw = _TRPW * 8
    mesh = plsc.VectorSubcoreMesh(core_axis_name="c", subcore_axis_name="s")
    lanes_out = jax.ShapeDtypeStruct((_N_SC * 16,), jnp.float32)

    @functools.partial(
        pl.kernel,
        mesh=mesh,
        out_type=(lanes_out, lanes_out, lanes_out, lanes_out),
        scratch_types=[
            pltpu.VMEM((8, _CHUNK), jnp.float32),
            pltpu.VMEM((8, 128), jnp.float32),
            pltpu.VMEM((max(rpw, 16),), jnp.int32),
            pltpu.VMEM((rpw * 16,), jnp.float32),
            pltpu.VMEM((rpw * 16,), jnp.float32),
            pltpu.VMEM((rpw * 16,), jnp.float32),
            pltpu.VMEM((rpw * 16,), jnp.float32),
        ],
        compiler_params=pltpu.CompilerParams(use_tc_tiling_on_sc=True),
    )
    def sc_stats(pred_hbm, tgt_hbm, m_hbm, s_hbm, sx_hbm, pt_hbm,
                 chunk_v, ttile_v, tgt_v, m_st, s_st, sx_st, pt_st):
        nc = 2
        wid = lax.axis_index("s") * nc + lax.axis_index("c")
        base = wid * rpw
        pltpu.sync_copy(tgt_hbm.at[pl.ds(base, max(rpw, 16))], tgt_v)
        lane = lax.broadcasted_iota(jnp.int32, (16,), 0)
        zeros = jnp.zeros((16,), jnp.float32)
        ninf = jnp.full((16,), -jnp.inf, jnp.float32)

        for tr in range(_TRPW):
            row0 = base + tr * 8

            def chunk_body(c, carry):
                ms, ss, sxs = carry
                pltpu.sync_copy(
                    pred_hbm.at[pl.ds(row0, 8), pl.ds(c * _CHUNK, _CHUNK)],
                    chunk_v)
                ms_n, ss_n, sxs_n = [], [], []
                for r8 in range(8):
                    def b1(i, mv, r8=r8):
                        for g in range(8):
                            mv = jnp.maximum(
                                mv, chunk_v[r8, pl.ds(i * 128 + g * 16, 16)])
                        return mv
                    cm = lax.fori_loop(0, ntiles, b1, ninf)
                    m_new = jnp.maximum(ms[r8], cm)

                    def b2(i, car, r8=r8, m_new=m_new):
                        sv, sxv = car
                        for g in range(8):
                            v = chunk_v[r8, pl.ds(i * 128 + g * 16, 16)]
                            sv = sv + jnp.exp(v - m_new)
                            sxv = sxv + v
                        return sv, sxv
                    sv, sxv = lax.fori_loop(0, ntiles, b2, (zeros, zeros))
                    ss_n.append(ss[r8] * jnp.exp(ms[r8] - m_new) + sv)
                    ms_n.append(m_new)
                    sxs_n.append(sxs[r8] + sxv)
                return tuple(ms_n), tuple(ss_n), tuple(sxs_n)

            ms, ss, sxs = lax.fori_loop(
                0, nchunks, chunk_body,
                ((ninf,) * 8, (zeros,) * 8, (zeros,) * 8))
            for r8 in range(8):
                ridx = tr * 8 + r8
                m_st[pl.ds(ridx * 16, 16)] = ms[r8]
                s_st[pl.ds(ridx * 16, 16)] = ss[r8]
                sx_st[pl.ds(ridx * 16, 16)] = sxs[r8]

        tlo = tgt_v[pl.ds(0, 16)]
        thi = tgt_v[pl.ds(rpw - 16, 16)] if rpw > 16 else tlo
        for r in range(rpw):
            t = tlo[r] if r < 16 else thi[r - (rpw - 16)]
            row0 = base + (r // 8) * 8
            pt_st[pl.ds(r * 16, 16)] = zeros

            @pl.when(t < _SC_COLS)
            def _(t=t, r=r, row0=row0):
                pltpu.sync_copy(
                    pred_hbm.at[pl.ds(row0, 8),
                                pl.ds((t // 128) * 128, 128)], ttile_v)
                w = ttile_v[r % 8, pl.ds(((t % 128) // 16) * 16, 16)]
                pt_st[pl.ds(r * 16, 16)] = jnp.where(
                    lane == t % 16, w, 0.0)

        pltpu.sync_copy(m_st, m_hbm.at[pl.ds(base * 16, rpw * 16)])
        pltpu.sync_copy(s_st, s_hbm.at[pl.ds(base * 16, rpw * 16)])
        pltpu.sync_copy(sx_st, sx_hbm.at[pl.ds(base * 16, rpw * 16)])
        pltpu.sync_copy(pt_st, pt_hbm.at[pl.ds(base * 16, rpw * 16)])

    return sc_stats


def kernel(pred, target):
    n, vocab = pred.shape
    tgt = target.astype(jnp.int32)
    tgt2d = tgt.reshape(n, 1)
    eps = _SMOOTHING / (vocab - 2)
    conf = 1.0 - _SMOOTHING

    m, s, sx, pt = _make_sc_part(vocab)(pred, tgt)
    mt, st, sxt, ptt = _tc_tail(pred[:_N_SC, _SC_COLS:], tgt2d, vocab)
    tc_losses = _tc_main(pred, tgt2d, vocab)

    m = m.reshape(_N_SC, 16)
    s = s.reshape(_N_SC, 16)
    sx = sx.reshape(_N_SC, 16)
    mt = mt.reshape(_N_SC)
    st = st.reshape(_N_SC)
    sxt = sxt.reshape(_N_SC)
    ptt = ptt.reshape(_N_SC)
    row_max = jnp.maximum(jnp.max(m, axis=1), mt)
    row_s = (jnp.sum(s * jnp.exp(m - row_max[:, None]), axis=1)
             + st * jnp.exp(mt - row_max))
    lse = row_max + jnp.log(row_s)
    sum_x = jnp.sum(sx, axis=1) + sxt
    pt_row = jnp.sum(pt.reshape(_N_SC, 16), axis=1) + ptt
    sc_loss = -(eps * (sum_x - vocab * lse) + (conf - eps) * (pt_row - lse))
    sc_loss = jnp.where(tgt[:_N_SC] == _IGNORE_INDEX, 0.0, sc_loss)
    return (jnp.sum(sc_loss) + jnp.sum(tc_losses)) / n
